# baseline (speedup 1.0000x reference)
import functools

import jax
import jax.numpy as jnp
from jax import lax
from jax.experimental import pallas as pl
from jax.experimental.pallas import tpu as pltpu
from jax.experimental.pallas import tpu_sc as plsc

L = 131072
_INFO = plsc.get_sparse_core_info()
NC = _INFO.num_cores
NS = _INFO.num_subcores
LANES = _INFO.num_lanes
NW = NC * NS
CHUNK = L // NW
NSUB = 4
SUB = CHUNK // NSUB


def _body(seq_hbm, out_hbm, idx_v, rows_v, sem_in, sem_out):
    wid = lax.axis_index("s") * NC + lax.axis_index("c")
    base = wid * CHUNK

    in_copies = [
        pltpu.async_copy(
            seq_hbm.at[pl.ds(base + s * SUB, SUB)], idx_v.at[s], sem_in
        )
        for s in range(NSUB)
    ]

    out_copies = []
    for s in range(NSUB):
        in_copies[s].wait()

        def step(i, carry, s=s):
            off = i * LANES
            v = idx_v[s, pl.ds(off, LANES)]
            fn = jnp.where(v == 4, jnp.float32(0.25), jnp.float32(0.0))
            for c in range(4):
                rows_v[c, pl.ds(s * SUB + off, LANES)] = jnp.where(
                    v == c, jnp.float32(1.0), fn
                )
            return carry

        lax.fori_loop(0, SUB // LANES, step, 0)

        for c in range(4):
            out_copies.append(
                pltpu.async_copy(
                    rows_v.at[c, pl.ds(s * SUB, SUB)],
                    out_hbm.at[c, pl.ds(base + s * SUB, SUB)],
                    sem_out,
                )
            )

    for cp in out_copies:
        cp.wait()


@jax.jit
def _seq2tensor(seq):
    run = functools.partial(
        pl.kernel,
        out_type=jax.ShapeDtypeStruct((4, L), jnp.float32),
        mesh=plsc.VectorSubcoreMesh(core_axis_name="c", subcore_axis_name="s"),
        scratch_types=[
            pltpu.VMEM((NSUB, SUB), jnp.int32),
            pltpu.VMEM((4, CHUNK), jnp.float32),
            pltpu.SemaphoreType.DMA,
            pltpu.SemaphoreType.DMA,
        ],
    )(_body)
    return run(seq)


def kernel(seq):
    return _seq2tensor(seq.astype(jnp.int32))
